# Initial kernel scaffold; baseline (speedup 1.0000x reference)
import functools

import jax
import jax.numpy as jnp
from jax import lax
from jax.experimental import pallas as pl
from jax.experimental.pallas import tpu as pltpu
from jax.experimental.pallas import tpu_sc as plsc

N = 50000
D = 128
K = 7
B = 50
NH = 4

QB = 1024
CT = 512
QB2 = 512
NPAD = 50176

SC_NC = 2
SC_NS = 16
SC_NW = SC_NC * SC_NS
SC_CH = 128


def _knn_body(posq_ref, qb_ref, ws_ref, nt_ref, posT_any, brow_any, nb_ref,
              cpos_ref, cbat_ref, bestd_ref, besti_ref, psem, bsem):
    i = pl.program_id(0)
    q = posq_ref[...]
    qn = jnp.sum(q * q, axis=1, keepdims=True)
    qb = qb_ref[...]
    qidx = i * QB + lax.broadcasted_iota(jnp.int32, (QB, 1), 0)
    bestd_ref[...] = jnp.full((QB, 8), jnp.inf, jnp.float32)
    besti_ref[...] = jnp.zeros((QB, 8), jnp.int32)
    ws = ws_ref[i]
    nt = nt_ref[i]
    lane = lax.broadcasted_iota(jnp.int32, (QB, 8), 1)

    def start_copy(t, slot):
        c0 = ws + t * CT
        pltpu.make_async_copy(posT_any.at[:, pl.ds(c0, CT)],
                              cpos_ref.at[slot], psem.at[slot]).start()
        pltpu.make_async_copy(brow_any.at[:, pl.ds(c0, CT)],
                              cbat_ref.at[slot], bsem.at[slot]).start()

    start_copy(0, 0)

    def body(t, _):
        slot = lax.rem(t, 2)
        nslot = lax.rem(t + 1, 2)

        @pl.when(t + 1 < nt)
        def _():
            start_copy(t + 1, nslot)

        c0 = ws + t * CT
        pltpu.make_async_copy(posT_any.at[:, pl.ds(c0, CT)],
                              cpos_ref.at[slot], psem.at[slot]).wait()
        pltpu.make_async_copy(brow_any.at[:, pl.ds(c0, CT)],
                              cbat_ref.at[slot], bsem.at[slot]).wait()
        c = cpos_ref[slot]
        cn = jnp.sum(c * c, axis=0, keepdims=True)
        d = qn + cn - 2.0 * jnp.dot(q, c, preferred_element_type=jnp.float32)
        cbat = cbat_ref[slot]
        cidx = c0 + lax.broadcasted_iota(jnp.int32, (1, CT), 1)
        d = jnp.where(qb != cbat, jnp.inf, d)
        d = jnp.where(qidx == cidx, jnp.inf, d)

        bd = bestd_ref[...]
        bi = besti_ref[...]
        for _sel in range(K):
            m = jnp.min(d, axis=1, keepdims=True)
            midx = jnp.min(jnp.where(d == m, cidx, jnp.int32(2**31 - 1)),
                           axis=1, keepdims=True)
            d = jnp.where(cidx == midx, jnp.inf, d)
            bd7 = jnp.where(lane < K, bd, -jnp.inf)
            w = jnp.max(bd7, axis=1, keepdims=True)
            fl = jnp.min(jnp.where(bd7 == w, lane, 8), axis=1, keepdims=True)
            sel = (lane == fl) & (m < w)
            bd = jnp.where(sel, m, bd)
            bi = jnp.where(sel, midx, bi)
        bestd_ref[...] = bd
        besti_ref[...] = bi
        return 0

    lax.fori_loop(0, nt, body, 0)
    nb_ref[...] = jnp.where(lane == K, qidx, besti_ref[...])


def _knn(pos4, batch_col, batch_row, ws_al, wcnt):
    nq = NPAD // QB
    return pl.pallas_call(
        _knn_body,
        grid=(nq,),
        in_specs=[
            pl.BlockSpec((QB, 4), lambda i: (i, 0)),
            pl.BlockSpec((QB, 1), lambda i: (i, 0)),
            pl.BlockSpec(memory_space=pltpu.SMEM),
            pl.BlockSpec(memory_space=pltpu.SMEM),
            pl.BlockSpec(memory_space=pltpu.ANY),
            pl.BlockSpec(memory_space=pltpu.ANY),
        ],
        out_specs=pl.BlockSpec((QB, 8), lambda i: (i, 0)),
        out_shape=jax.ShapeDtypeStruct((NPAD, 8), jnp.int32),
        scratch_shapes=[
            pltpu.VMEM((2, 4, CT), jnp.float32),
            pltpu.VMEM((2, 1, CT), jnp.int32),
            pltpu.VMEM((QB, 8), jnp.float32),
            pltpu.VMEM((QB, 8), jnp.int32),
            pltpu.SemaphoreType.DMA((2,)),
            pltpu.SemaphoreType.DMA((2,)),
        ],
        compiler_params=pltpu.CompilerParams(
            dimension_semantics=("arbitrary",)),
    )(pos4, batch_col, ws_al, wcnt, pos4.T, batch_row)


def _proj_body(x_ref, wl_ref, wr_ref, xl_ref, xr_ref):
    xv = x_ref[...]
    xl_ref[...] = jnp.dot(xv, wl_ref[...], preferred_element_type=jnp.float32)
    xr_ref[...] = jnp.dot(xv, wr_ref[...], preferred_element_type=jnp.float32)


def _proj(xin, Wl, Wr):
    nb2 = NPAD // QB2
    din, c = Wl.shape
    return pl.pallas_call(
        _proj_body,
        grid=(nb2,),
        in_specs=[
            pl.BlockSpec((QB2, din), lambda i: (i, 0)),
            pl.BlockSpec((din, c), lambda i: (0, 0)),
            pl.BlockSpec((din, c), lambda i: (0, 0)),
        ],
        out_specs=[
            pl.BlockSpec((QB2, c), lambda i: (i, 0)),
            pl.BlockSpec((QB2, c), lambda i: (i, 0)),
        ],
        out_shape=[
            jax.ShapeDtypeStruct((NPAD, c), jnp.float32),
            jax.ShapeDtypeStruct((NPAD, c), jnp.float32),
        ],
    )(xin, Wl, Wr)


def _gather_rows(table, idx, etotp, c):
    pw = etotp // SC_NW
    nch = pw // SC_CH
    mesh = plsc.VectorSubcoreMesh(core_axis_name="c", subcore_axis_name="s")

    @functools.partial(
        pl.kernel,
        mesh=mesh,
        out_type=jax.ShapeDtypeStruct((etotp, c), jnp.float32),
        scratch_types=[
            pltpu.VMEM((pw,), jnp.int32),
            pltpu.VMEM((SC_CH, c), jnp.float32),
            pltpu.SemaphoreType.DMA,
        ],
    )
    def gat_gather(table_hbm, idx_hbm, out_hbm, idx_v, rows_v, sem):
        wid = lax.axis_index("s") * SC_NC + lax.axis_index("c")
        base = wid * pw
        pltpu.sync_copy(idx_hbm.at[pl.ds(base, pw)], idx_v)

        def body(ci, _):
            off = ci * SC_CH
            pltpu.async_copy(table_hbm.at[idx_v.at[pl.ds(off, SC_CH)]],
                             rows_v, sem).wait()
            pltpu.sync_copy(rows_v, out_hbm.at[pl.ds(base + off, SC_CH)])
            return 0

        lax.fori_loop(0, nch, body, 0)

    return gat_gather(table, idx)


def _attn_body(g_ref, xr_ref, att_ref, bias_ref, h_ref, *, ch):
    c = NH * ch
    xr = xr_ref[...]
    att = att_ref[...]
    S = (lax.broadcasted_iota(jnp.int32, (c, NH), 0) // ch
         == lax.broadcasted_iota(jnp.int32, (c, NH), 1)).astype(jnp.float32)
    ST = (lax.broadcasted_iota(jnp.int32, (NH, c), 0)
          == lax.broadcasted_iota(jnp.int32, (NH, c), 1) // ch
          ).astype(jnp.float32)

    a = []
    for j in range(K):
        v = g_ref[j] + xr
        e = jnp.where(v >= 0, v, 0.2 * v)
        a.append(jnp.dot(e * att, S, preferred_element_type=jnp.float32))
    amax = a[0]
    for j in range(1, K):
        amax = jnp.maximum(amax, a[j])
    ex = [jnp.exp(a[j] - amax) for j in range(K)]
    denom = ex[0]
    for j in range(1, K):
        denom = denom + ex[j]
    out = jnp.zeros((QB2, c), jnp.float32)
    for j in range(K):
        w = ex[j] / (denom + 1e-16)
        wex = jnp.dot(w, ST, preferred_element_type=jnp.float32)
        out = out + g_ref[j] * wex
    h_ref[...] = jnp.maximum(out + bias_ref[...], 0.0)


def _attn(g, xr, att, bias, ch):
    nb2 = NPAD // QB2
    c = NH * ch
    return pl.pallas_call(
        functools.partial(_attn_body, ch=ch),
        grid=(nb2,),
        in_specs=[
            pl.BlockSpec((K, QB2, c), lambda i: (0, i, 0)),
            pl.BlockSpec((QB2, c), lambda i: (i, 0)),
            pl.BlockSpec((1, c), lambda i: (0, 0)),
            pl.BlockSpec((1, c), lambda i: (0, 0)),
        ],
        out_specs=pl.BlockSpec((QB2, c), lambda i: (i, 0)),
        out_shape=jax.ShapeDtypeStruct((NPAD, c), jnp.float32),
    )(g, xr, att, bias)


BPAD = 56


def _pool_body(h_ref, brow_ref, out_ref, acc_ref, cnt_ref):
    i = pl.program_id(0)
    nb2 = pl.num_programs(0)

    @pl.when(i == 0)
    def _():
        acc_ref[...] = jnp.zeros((BPAD, D), jnp.float32)
        cnt_ref[...] = jnp.zeros((BPAD, 1), jnp.float32)

    br = brow_ref[...]
    oh = (lax.broadcasted_iota(jnp.int32, (BPAD, QB2), 0) == br
          ).astype(jnp.float32)
    acc_ref[...] += jnp.dot(oh, h_ref[...], preferred_element_type=jnp.float32)
    cnt_ref[...] += jnp.sum(oh, axis=1, keepdims=True)

    @pl.when(i == nb2 - 1)
    def _():
        out_ref[...] = acc_ref[...] / jnp.maximum(cnt_ref[...], 1.0)


def _pool(h, batch_row):
    nb2 = NPAD // QB2
    return pl.pallas_call(
        _pool_body,
        grid=(nb2,),
        in_specs=[
            pl.BlockSpec((QB2, D), lambda i: (i, 0)),
            pl.BlockSpec((1, QB2), lambda i: (0, i)),
        ],
        out_specs=pl.BlockSpec((BPAD, D), lambda i: (0, 0)),
        out_shape=jax.ShapeDtypeStruct((BPAD, D), jnp.float32),
        scratch_shapes=[
            pltpu.VMEM((BPAD, D), jnp.float32),
            pltpu.VMEM((BPAD, 1), jnp.float32),
        ],
        compiler_params=pltpu.CompilerParams(
            dimension_semantics=("arbitrary",)),
    )(h, batch_row)


def kernel(x, pos, batch, Wl1, Wr1, att1, bias1, Wl2, Wr2, att2, bias2):
    pad = NPAD - N
    pos4 = jnp.pad(pos, ((0, pad), (0, 1)))
    batch_p = jnp.pad(batch, (0, pad), constant_values=-1)
    batch_col = batch_p.reshape(NPAD, 1)
    batch_row = batch_p.reshape(1, NPAD)
    x_p = jnp.pad(x, ((0, pad), (0, 0)))

    nq = NPAD // QB
    starts = jnp.arange(nq) * QB
    qfirst = batch_p[starts]
    qlast = batch_p[jnp.minimum(starts + QB - 1, N - 1)]
    wstart = jnp.searchsorted(batch, qfirst, side="left").astype(jnp.int32)
    wend = jnp.searchsorted(batch, qlast, side="right").astype(jnp.int32)
    ws_al = (wstart // CT) * CT
    wcnt = (wend - ws_al + CT - 1) // CT

    nb = _knn(pos4, batch_col, batch_row, ws_al, wcnt)

    etot = K * NPAD
    etotp = ((etot + SC_NW * SC_CH - 1) // (SC_NW * SC_CH)) * (SC_NW * SC_CH)
    idx = nb[:, :K].T.reshape(-1)
    idx = jnp.pad(idx, (0, etotp - etot))

    xl1, xr1 = _proj(x_p, Wl1, Wr1)
    g1 = _gather_rows(xl1, idx, etotp, 64)[:etot].reshape(K, NPAD, 64)
    h1 = _attn(g1, xr1, att1.reshape(1, 64), bias1.reshape(1, 64), 16)

    xl2, xr2 = _proj(h1, Wl2, Wr2)
    g2 = _gather_rows(xl2, idx, etotp, 128)[:etot].reshape(K, NPAD, 128)
    h2 = _attn(g2, xr2, att2.reshape(1, 128), bias2.reshape(1, 128), 32)

    pooled = _pool(h2, batch_row)
    return pooled[:B]


# speedup vs baseline: 48.1014x; 48.1014x over previous
import functools

import jax
import jax.numpy as jnp
from jax import lax
from jax.experimental import pallas as pl
from jax.experimental.pallas import tpu as pltpu
from jax.experimental.pallas import tpu_sc as plsc

N = 50000
D = 128
K = 7
B = 50
NH = 4

QB = 1024
CT = 512
QB2 = 512
NPAD = 50176

SC_NC = 2
SC_NS = 16
SC_NW = SC_NC * SC_NS
SC_CH = 128


def _knn_body(posq_ref, qb_ref, ws_ref, nt_ref, posT_any, brow_any, nb_ref,
              cpos_ref, cbat_ref, bestd_ref, besti_ref, psem, bsem):
    i = pl.program_id(0)
    q = posq_ref[...]
    qn = jnp.sum(q * q, axis=1, keepdims=True)
    qb = qb_ref[...]
    qidx = i * QB + lax.broadcasted_iota(jnp.int32, (QB, 1), 0)
    bestd_ref[...] = jnp.full((QB, 8), jnp.inf, jnp.float32)
    besti_ref[...] = jnp.zeros((QB, 8), jnp.int32)
    ws = ws_ref[i]
    nt = nt_ref[i]
    lane = lax.broadcasted_iota(jnp.int32, (QB, 8), 1)

    def start_copy(t, slot):
        c0 = pl.multiple_of(ws + t * CT, CT)
        pltpu.make_async_copy(posT_any.at[:, pl.ds(c0, CT)],
                              cpos_ref.at[slot], psem.at[slot]).start()
        pltpu.make_async_copy(brow_any.at[:, pl.ds(c0, CT)],
                              cbat_ref.at[slot], bsem.at[slot]).start()

    start_copy(0, 0)

    def body(t, _):
        slot = lax.rem(t, 2)
        nslot = lax.rem(t + 1, 2)

        @pl.when(t + 1 < nt)
        def _():
            start_copy(t + 1, nslot)

        c0 = pl.multiple_of(ws + t * CT, CT)
        pltpu.make_async_copy(posT_any.at[:, pl.ds(c0, CT)],
                              cpos_ref.at[slot], psem.at[slot]).wait()
        pltpu.make_async_copy(brow_any.at[:, pl.ds(c0, CT)],
                              cbat_ref.at[slot], bsem.at[slot]).wait()
        c = cpos_ref[slot]
        cn = jnp.sum(c * c, axis=0, keepdims=True)
        d = qn + cn - 2.0 * jnp.dot(q, c, preferred_element_type=jnp.float32)
        cbat = cbat_ref[slot]
        cidx = c0 + lax.broadcasted_iota(jnp.int32, (1, CT), 1)
        d = jnp.where(qb != cbat, jnp.inf, d)
        d = jnp.where(qidx == cidx, jnp.inf, d)

        bd = bestd_ref[...]
        bi = besti_ref[...]
        for _sel in range(K):
            m = jnp.min(d, axis=1, keepdims=True)
            midx = jnp.min(jnp.where(d == m, cidx, jnp.int32(2**31 - 1)),
                           axis=1, keepdims=True)
            d = jnp.where(cidx == midx, jnp.inf, d)
            bd7 = jnp.where(lane < K, bd, -jnp.inf)
            w = jnp.max(bd7, axis=1, keepdims=True)
            fl = jnp.min(jnp.where(bd7 == w, lane, 8), axis=1, keepdims=True)
            sel = (lane == fl) & (m < w)
            bd = jnp.where(sel, m, bd)
            bi = jnp.where(sel, midx, bi)
        bestd_ref[...] = bd
        besti_ref[...] = bi
        return 0

    lax.fori_loop(0, nt, body, 0)
    nb_ref[...] = jnp.where(lane == K, qidx, besti_ref[...])


def _knn(pos4, batch_col, batch_row, ws_al, wcnt):
    nq = NPAD // QB
    return pl.pallas_call(
        _knn_body,
        grid=(nq,),
        in_specs=[
            pl.BlockSpec((QB, 4), lambda i: (i, 0)),
            pl.BlockSpec((QB, 1), lambda i: (i, 0)),
            pl.BlockSpec(memory_space=pltpu.SMEM),
            pl.BlockSpec(memory_space=pltpu.SMEM),
            pl.BlockSpec(memory_space=pl.ANY),
            pl.BlockSpec(memory_space=pl.ANY),
        ],
        out_specs=pl.BlockSpec((QB, 8), lambda i: (i, 0)),
        out_shape=jax.ShapeDtypeStruct((NPAD, 8), jnp.int32),
        scratch_shapes=[
            pltpu.VMEM((2, 4, CT), jnp.float32),
            pltpu.VMEM((2, 1, CT), jnp.int32),
            pltpu.VMEM((QB, 8), jnp.float32),
            pltpu.VMEM((QB, 8), jnp.int32),
            pltpu.SemaphoreType.DMA((2,)),
            pltpu.SemaphoreType.DMA((2,)),
        ],
        compiler_params=pltpu.CompilerParams(
            dimension_semantics=("arbitrary",)),
    )(pos4, batch_col, ws_al, wcnt, pos4.T, batch_row)


def _proj_body(x_ref, wl_ref, wr_ref, xl_ref, xr_ref):
    xv = x_ref[...]
    xl_ref[...] = jnp.dot(xv, wl_ref[...], preferred_element_type=jnp.float32)
    xr_ref[...] = jnp.dot(xv, wr_ref[...], preferred_element_type=jnp.float32)


def _proj(xin, Wl, Wr):
    nb2 = NPAD // QB2
    din, cl = Wl.shape
    cr = Wr.shape[1]
    return pl.pallas_call(
        _proj_body,
        grid=(nb2,),
        in_specs=[
            pl.BlockSpec((QB2, din), lambda i: (i, 0)),
            pl.BlockSpec((din, cl), lambda i: (0, 0)),
            pl.BlockSpec((din, cr), lambda i: (0, 0)),
        ],
        out_specs=[
            pl.BlockSpec((QB2, cl), lambda i: (i, 0)),
            pl.BlockSpec((QB2, cr), lambda i: (i, 0)),
        ],
        out_shape=[
            jax.ShapeDtypeStruct((NPAD, cl), jnp.float32),
            jax.ShapeDtypeStruct((NPAD, cr), jnp.float32),
        ],
    )(xin, Wl, Wr)


def _gather_rows(table, idx, etotp, c):
    pw = etotp // SC_NW
    nch = pw // SC_CH
    mesh = plsc.VectorSubcoreMesh(core_axis_name="c", subcore_axis_name="s")

    @functools.partial(
        pl.kernel,
        mesh=mesh,
        out_type=jax.ShapeDtypeStruct((etotp, c), jnp.float32),
        scratch_types=[
            pltpu.VMEM((pw,), jnp.int32),
            pltpu.VMEM((SC_CH, c), jnp.float32),
            pltpu.SemaphoreType.DMA,
        ],
    )
    def gat_gather(table_hbm, idx_hbm, out_hbm, idx_v, rows_v, sem):
        wid = lax.axis_index("s") * SC_NC + lax.axis_index("c")
        base = wid * pw
        pltpu.sync_copy(idx_hbm.at[pl.ds(base, pw)], idx_v)

        def body(ci, _):
            off = ci * SC_CH
            pltpu.async_copy(table_hbm.at[idx_v.at[pl.ds(off, SC_CH)]],
                             rows_v, sem).wait()
            pltpu.sync_copy(rows_v, out_hbm.at[pl.ds(base + off, SC_CH)])
            return 0

        lax.fori_loop(0, nch, body, 0)

    return gat_gather(table, idx)


def _attn_body(g_ref, xr_ref, att_ref, bias_ref, h_ref, *, ch):
    c = NH * ch
    xr = xr_ref[...]
    gs = [g_ref[j][:, :c] for j in range(K)]
    att = att_ref[...]
    S = (lax.broadcasted_iota(jnp.int32, (c, NH), 0) // ch
         == lax.broadcasted_iota(jnp.int32, (c, NH), 1)).astype(jnp.float32)
    ST = (lax.broadcasted_iota(jnp.int32, (NH, c), 0)
          == lax.broadcasted_iota(jnp.int32, (NH, c), 1) // ch
          ).astype(jnp.float32)

    a = []
    for j in range(K):
        v = gs[j] + xr
        e = jnp.where(v >= 0, v, 0.2 * v)
        a.append(jnp.dot(e * att, S, preferred_element_type=jnp.float32))
    amax = a[0]
    for j in range(1, K):
        amax = jnp.maximum(amax, a[j])
    ex = [jnp.exp(a[j] - amax) for j in range(K)]
    denom = ex[0]
    for j in range(1, K):
        denom = denom + ex[j]
    out = jnp.zeros((QB2, c), jnp.float32)
    for j in range(K):
        w = ex[j] / (denom + 1e-16)
        wex = jnp.dot(w, ST, preferred_element_type=jnp.float32)
        out = out + gs[j] * wex
    h_ref[...] = jnp.maximum(out + bias_ref[...], 0.0)


def _attn(g, xr, att, bias, ch):
    nb2 = NPAD // QB2
    c = NH * ch
    gc = g.shape[-1]
    return pl.pallas_call(
        functools.partial(_attn_body, ch=ch),
        grid=(nb2,),
        in_specs=[
            pl.BlockSpec((K, QB2, gc), lambda i: (0, i, 0)),
            pl.BlockSpec((QB2, c), lambda i: (i, 0)),
            pl.BlockSpec((1, c), lambda i: (0, 0)),
            pl.BlockSpec((1, c), lambda i: (0, 0)),
        ],
        out_specs=pl.BlockSpec((QB2, c), lambda i: (i, 0)),
        out_shape=jax.ShapeDtypeStruct((NPAD, c), jnp.float32),
    )(g, xr, att, bias)


BPAD = 56


def _pool_body(h_ref, brow_ref, out_ref, acc_ref, cnt_ref):
    i = pl.program_id(0)
    nb2 = pl.num_programs(0)

    @pl.when(i == 0)
    def _():
        acc_ref[...] = jnp.zeros((BPAD, D), jnp.float32)
        cnt_ref[...] = jnp.zeros((BPAD, 1), jnp.float32)

    br = brow_ref[...]
    oh = (lax.broadcasted_iota(jnp.int32, (BPAD, QB2), 0) == br
          ).astype(jnp.float32)
    acc_ref[...] += jnp.dot(oh, h_ref[...], preferred_element_type=jnp.float32)
    cnt_ref[...] += jnp.sum(oh, axis=1, keepdims=True)

    @pl.when(i == nb2 - 1)
    def _():
        out_ref[...] = acc_ref[...] / jnp.maximum(cnt_ref[...], 1.0)


def _pool(h, batch_row):
    nb2 = NPAD // QB2
    return pl.pallas_call(
        _pool_body,
        grid=(nb2,),
        in_specs=[
            pl.BlockSpec((QB2, D), lambda i: (i, 0)),
            pl.BlockSpec((1, QB2), lambda i: (0, i)),
        ],
        out_specs=pl.BlockSpec((BPAD, D), lambda i: (0, 0)),
        out_shape=jax.ShapeDtypeStruct((BPAD, D), jnp.float32),
        scratch_shapes=[
            pltpu.VMEM((BPAD, D), jnp.float32),
            pltpu.VMEM((BPAD, 1), jnp.float32),
        ],
        compiler_params=pltpu.CompilerParams(
            dimension_semantics=("arbitrary",)),
    )(h, batch_row)


def kernel(x, pos, batch, Wl1, Wr1, att1, bias1, Wl2, Wr2, att2, bias2):
    pad = NPAD - N
    pos4 = jnp.pad(pos, ((0, pad), (0, 1)))
    batch_p = jnp.pad(batch, (0, pad), constant_values=-1)
    batch_col = batch_p.reshape(NPAD, 1)
    batch_row = batch_p.reshape(1, NPAD)
    x_p = jnp.pad(x, ((0, pad), (0, 0)))

    nq = NPAD // QB
    starts = jnp.arange(nq) * QB
    qfirst = batch_p[starts]
    qlast = batch_p[jnp.minimum(starts + QB - 1, N - 1)]
    wstart = jnp.searchsorted(batch, qfirst, side="left").astype(jnp.int32)
    wend = jnp.searchsorted(batch, qlast, side="right").astype(jnp.int32)
    ws_al = (wstart // CT) * CT
    wcnt = (wend - ws_al + CT - 1) // CT

    nb = _knn(pos4, batch_col, batch_row, ws_al, wcnt)

    etot = K * NPAD
    etotp = ((etot + SC_NW * SC_CH - 1) // (SC_NW * SC_CH)) * (SC_NW * SC_CH)
    idx = nb[:, :K].T.reshape(-1)
    idx = jnp.pad(idx, (0, etotp - etot))

    Wl1p = jnp.pad(Wl1, ((0, 0), (0, D - 64)))
    xl1, xr1 = _proj(x_p, Wl1p, Wr1)
    g1 = _gather_rows(xl1, idx, etotp, 128)[:etot].reshape(K, NPAD, 128)
    h1 = _attn(g1, xr1, att1.reshape(1, 64), bias1.reshape(1, 64), 16)

    xl2, xr2 = _proj(h1, Wl2, Wr2)
    g2 = _gather_rows(xl2, idx, etotp, 128)[:etot].reshape(K, NPAD, 128)
    h2 = _attn(g2, xr2, att2.reshape(1, 128), bias2.reshape(1, 128), 32)

    pooled = _pool(h2, batch_row)
    return pooled[:B]


# speedup vs baseline: 57.3767x; 1.1928x over previous
import functools

import jax
import jax.numpy as jnp
from jax import lax
from jax.experimental import pallas as pl
from jax.experimental.pallas import tpu as pltpu
from jax.experimental.pallas import tpu_sc as plsc

N = 50000
D = 128
K = 7
B = 50
NH = 4

QB = 512
CT = 512
QB2 = 512
NPAD = 50176

SC_NC = 2
SC_NS = 16
SC_NW = SC_NC * SC_NS
SC_CH = 112


def _knn_body(posq_ref, qb_ref, ws_ref, nt_ref, posT_any, brow_any, nb_ref,
              cpos_ref, cbat_ref, bestd_ref, besti_ref, psem, bsem):
    i = pl.program_id(0)
    q = posq_ref[...]
    qn = jnp.sum(q * q, axis=1, keepdims=True)
    qb = qb_ref[...]
    qidx = i * QB + lax.broadcasted_iota(jnp.int32, (QB, 1), 0)
    bestd_ref[...] = jnp.full((QB, 8), jnp.inf, jnp.float32)
    besti_ref[...] = jnp.zeros((QB, 8), jnp.int32)
    ws = ws_ref[i]
    nt = nt_ref[i]
    lane = lax.broadcasted_iota(jnp.int32, (QB, 8), 1)

    def start_copy(t, slot):
        c0 = pl.multiple_of(ws + t * CT, CT)
        pltpu.make_async_copy(posT_any.at[:, pl.ds(c0, CT)],
                              cpos_ref.at[slot], psem.at[slot]).start()
        pltpu.make_async_copy(brow_any.at[:, pl.ds(c0, CT)],
                              cbat_ref.at[slot], bsem.at[slot]).start()

    start_copy(0, 0)

    def body(t, _):
        slot = lax.rem(t, 2)
        nslot = lax.rem(t + 1, 2)

        @pl.when(t + 1 < nt)
        def _():
            start_copy(t + 1, nslot)

        c0 = pl.multiple_of(ws + t * CT, CT)
        pltpu.make_async_copy(posT_any.at[:, pl.ds(c0, CT)],
                              cpos_ref.at[slot], psem.at[slot]).wait()
        pltpu.make_async_copy(brow_any.at[:, pl.ds(c0, CT)],
                              cbat_ref.at[slot], bsem.at[slot]).wait()
        c = cpos_ref[slot]
        cn = jnp.sum(c * c, axis=0, keepdims=True)
        d = qn + cn - 2.0 * jnp.dot(q, c, preferred_element_type=jnp.float32)
        cbat = cbat_ref[slot]
        cidx = c0 + lax.broadcasted_iota(jnp.int32, (1, CT), 1)
        d = jnp.where(qb != cbat, jnp.inf, d)
        d = jnp.where(qidx == cidx, jnp.inf, d)

        bd = bestd_ref[...]
        bi = besti_ref[...]
        for _sel in range(K):
            m = jnp.min(d, axis=1, keepdims=True)
            midx = jnp.min(jnp.where(d == m, cidx, jnp.int32(2**31 - 1)),
                           axis=1, keepdims=True)
            d = jnp.where(cidx == midx, jnp.inf, d)
            bd7 = jnp.where(lane < K, bd, -jnp.inf)
            w = jnp.max(bd7, axis=1, keepdims=True)
            fl = jnp.min(jnp.where(bd7 == w, lane, 8), axis=1, keepdims=True)
            sel = (lane == fl) & (m < w)
            bd = jnp.where(sel, m, bd)
            bi = jnp.where(sel, midx, bi)
        bestd_ref[...] = bd
        besti_ref[...] = bi
        return 0

    lax.fori_loop(0, nt, body, 0)
    nb_ref[...] = jnp.where(lane == K, qidx, besti_ref[...])


def _knn(pos4, batch_col, batch_row, ws_al, wcnt):
    nq = NPAD // QB
    return pl.pallas_call(
        _knn_body,
        grid=(nq,),
        in_specs=[
            pl.BlockSpec((QB, 4), lambda i: (i, 0)),
            pl.BlockSpec((QB, 1), lambda i: (i, 0)),
            pl.BlockSpec(memory_space=pltpu.SMEM),
            pl.BlockSpec(memory_space=pltpu.SMEM),
            pl.BlockSpec(memory_space=pl.ANY),
            pl.BlockSpec(memory_space=pl.ANY),
        ],
        out_specs=pl.BlockSpec((QB, 8), lambda i: (i, 0)),
        out_shape=jax.ShapeDtypeStruct((NPAD, 8), jnp.int32),
        scratch_shapes=[
            pltpu.VMEM((2, 4, CT), jnp.float32),
            pltpu.VMEM((2, 1, CT), jnp.int32),
            pltpu.VMEM((QB, 8), jnp.float32),
            pltpu.VMEM((QB, 8), jnp.int32),
            pltpu.SemaphoreType.DMA((2,)),
            pltpu.SemaphoreType.DMA((2,)),
        ],
        compiler_params=pltpu.CompilerParams(
            dimension_semantics=("arbitrary",)),
    )(pos4, batch_col, ws_al, wcnt, pos4.T, batch_row)


def _proj_body(x_ref, wl_ref, wr_ref, xl_ref, xr_ref):
    xv = x_ref[...]
    xl_ref[...] = jnp.dot(xv, wl_ref[...], preferred_element_type=jnp.float32)
    xr_ref[...] = jnp.dot(xv, wr_ref[...], preferred_element_type=jnp.float32)


def _proj(xin, Wl, Wr):
    nb2 = NPAD // QB2
    din, cl = Wl.shape
    cr = Wr.shape[1]
    return pl.pallas_call(
        _proj_body,
        grid=(nb2,),
        in_specs=[
            pl.BlockSpec((QB2, din), lambda i: (i, 0)),
            pl.BlockSpec((din, cl), lambda i: (0, 0)),
            pl.BlockSpec((din, cr), lambda i: (0, 0)),
        ],
        out_specs=[
            pl.BlockSpec((QB2, cl), lambda i: (i, 0)),
            pl.BlockSpec((QB2, cr), lambda i: (i, 0)),
        ],
        out_shape=[
            jax.ShapeDtypeStruct((NPAD, cl), jnp.float32),
            jax.ShapeDtypeStruct((NPAD, cr), jnp.float32),
        ],
    )(xin, Wl, Wr)


def _gather_rows(table, idx, etotp, c):
    pw = etotp // SC_NW
    nch = pw // SC_CH
    mesh = plsc.VectorSubcoreMesh(core_axis_name="c", subcore_axis_name="s")

    @functools.partial(
        pl.kernel,
        mesh=mesh,
        out_type=jax.ShapeDtypeStruct((etotp, c), jnp.float32),
        scratch_types=[
            pltpu.VMEM((pw,), jnp.int32),
            pltpu.VMEM((2, SC_CH, c), jnp.float32),
            pltpu.SemaphoreType.DMA((2,)),
        ],
    )
    def gat_gather(table_hbm, idx_hbm, out_hbm, idx_v, rows_v, sem):
        wid = lax.axis_index("s") * SC_NC + lax.axis_index("c")
        base = wid * pw
        pltpu.sync_copy(idx_hbm.at[pl.ds(base, pw)], idx_v)

        def start_g(ci, slot):
            off = ci * SC_CH
            pltpu.make_async_copy(table_hbm.at[idx_v.at[pl.ds(off, SC_CH)]],
                                  rows_v.at[slot], sem.at[slot]).start()

        start_g(0, 0)

        def body(ci, _):
            slot = lax.rem(ci, 2)
            off = ci * SC_CH

            @pl.when(ci + 1 < nch)
            def _():
                start_g(ci + 1, lax.rem(ci + 1, 2))

            pltpu.make_async_copy(table_hbm.at[idx_v.at[pl.ds(off, SC_CH)]],
                                  rows_v.at[slot], sem.at[slot]).wait()
            pltpu.sync_copy(rows_v.at[slot],
                            out_hbm.at[pl.ds(base + off, SC_CH)])
            return 0

        lax.fori_loop(0, nch, body, 0)

    return gat_gather(table, idx)


def _attn_body(g_ref, xr_ref, att_ref, bias_ref, h_ref, *, ch):
    c = NH * ch
    xr = xr_ref[...]
    gs = [g_ref[j][:, :c] for j in range(K)]
    att = att_ref[...]
    S = (lax.broadcasted_iota(jnp.int32, (c, NH), 0) // ch
         == lax.broadcasted_iota(jnp.int32, (c, NH), 1)).astype(jnp.float32)
    ST = (lax.broadcasted_iota(jnp.int32, (NH, c), 0)
          == lax.broadcasted_iota(jnp.int32, (NH, c), 1) // ch
          ).astype(jnp.float32)

    a = []
    for j in range(K):
        v = gs[j] + xr
        e = jnp.where(v >= 0, v, 0.2 * v)
        a.append(jnp.dot(e * att, S, preferred_element_type=jnp.float32))
    amax = a[0]
    for j in range(1, K):
        amax = jnp.maximum(amax, a[j])
    ex = [jnp.exp(a[j] - amax) for j in range(K)]
    denom = ex[0]
    for j in range(1, K):
        denom = denom + ex[j]
    out = jnp.zeros((QB2, c), jnp.float32)
    for j in range(K):
        w = ex[j] / (denom + 1e-16)
        wex = jnp.dot(w, ST, preferred_element_type=jnp.float32)
        out = out + gs[j] * wex
    h_ref[...] = jnp.maximum(out + bias_ref[...], 0.0)


def _attn(g, xr, att, bias, ch):
    nb2 = NPAD // QB2
    c = NH * ch
    gc = g.shape[-1]
    return pl.pallas_call(
        functools.partial(_attn_body, ch=ch),
        grid=(nb2,),
        in_specs=[
            pl.BlockSpec((K, QB2, gc), lambda i: (0, i, 0)),
            pl.BlockSpec((QB2, c), lambda i: (i, 0)),
            pl.BlockSpec((1, c), lambda i: (0, 0)),
            pl.BlockSpec((1, c), lambda i: (0, 0)),
        ],
        out_specs=pl.BlockSpec((QB2, c), lambda i: (i, 0)),
        out_shape=jax.ShapeDtypeStruct((NPAD, c), jnp.float32),
    )(g, xr, att, bias)


BPAD = 56


def _pool_body(h_ref, brow_ref, out_ref, acc_ref, cnt_ref):
    i = pl.program_id(0)
    nb2 = pl.num_programs(0)

    @pl.when(i == 0)
    def _():
        acc_ref[...] = jnp.zeros((BPAD, D), jnp.float32)
        cnt_ref[...] = jnp.zeros((BPAD, 1), jnp.float32)

    br = brow_ref[...]
    oh = (lax.broadcasted_iota(jnp.int32, (BPAD, QB2), 0) == br
          ).astype(jnp.float32)
    acc_ref[...] += jnp.dot(oh, h_ref[...], preferred_element_type=jnp.float32)
    cnt_ref[...] += jnp.sum(oh, axis=1, keepdims=True)

    @pl.when(i == nb2 - 1)
    def _():
        out_ref[...] = acc_ref[...] / jnp.maximum(cnt_ref[...], 1.0)


def _pool(h, batch_row):
    nb2 = NPAD // QB2
    return pl.pallas_call(
        _pool_body,
        grid=(nb2,),
        in_specs=[
            pl.BlockSpec((QB2, D), lambda i: (i, 0)),
            pl.BlockSpec((1, QB2), lambda i: (0, i)),
        ],
        out_specs=pl.BlockSpec((BPAD, D), lambda i: (0, 0)),
        out_shape=jax.ShapeDtypeStruct((BPAD, D), jnp.float32),
        scratch_shapes=[
            pltpu.VMEM((BPAD, D), jnp.float32),
            pltpu.VMEM((BPAD, 1), jnp.float32),
        ],
        compiler_params=pltpu.CompilerParams(
            dimension_semantics=("arbitrary",)),
    )(h, batch_row)


def kernel(x, pos, batch, Wl1, Wr1, att1, bias1, Wl2, Wr2, att2, bias2):
    pad = NPAD - N
    pos4 = jnp.pad(pos, ((0, pad), (0, 1)))
    batch_p = jnp.pad(batch, (0, pad), constant_values=-1)
    batch_col = batch_p.reshape(NPAD, 1)
    batch_row = batch_p.reshape(1, NPAD)
    x_p = jnp.pad(x, ((0, pad), (0, 0)))

    nq = NPAD // QB
    starts = jnp.arange(nq) * QB
    qfirst = batch_p[starts]
    qlast = batch_p[jnp.minimum(starts + QB - 1, N - 1)]
    wstart = jnp.searchsorted(batch, qfirst, side="left").astype(jnp.int32)
    wend = jnp.searchsorted(batch, qlast, side="right").astype(jnp.int32)
    ws_al = (wstart // CT) * CT
    wcnt = (wend - ws_al + CT - 1) // CT

    nb = _knn(pos4, batch_col, batch_row, ws_al, wcnt)

    etot = K * NPAD
    idx = nb[:, :K].T.reshape(-1)

    Wl1p = jnp.pad(Wl1, ((0, 0), (0, D - 64)))
    xl1, xr1 = _proj(x_p, Wl1p, Wr1)
    g1 = _gather_rows(xl1, idx, etot, 128).reshape(K, NPAD, 128)
    h1 = _attn(g1, xr1, att1.reshape(1, 64), bias1.reshape(1, 64), 16)

    xl2, xr2 = _proj(h1, Wl2, Wr2)
    g2 = _gather_rows(xl2, idx, etot, 128).reshape(K, NPAD, 128)
    h2 = _attn(g2, xr2, att2.reshape(1, 128), bias2.reshape(1, 128), 32)

    pooled = _pool(h2, batch_row)
    return pooled[:B]


# speedup vs baseline: 66.2622x; 1.1549x over previous
import functools

import jax
import jax.numpy as jnp
from jax import lax
from jax.experimental import pallas as pl
from jax.experimental.pallas import tpu as pltpu
from jax.experimental.pallas import tpu_sc as plsc

N = 50000
D = 128
K = 7
B = 50
NH = 4

QB = 512
CT = 512
QB2 = 512
NPAD = 50176

SC_NC = 2
SC_NS = 16
SC_NW = SC_NC * SC_NS
SC_CH = 112


def _knn_body(posq_ref, qb_ref, ws_ref, nt_ref, posT_any, brow_any, nb_ref,
              cpos_ref, cbat_ref, bestd_ref, besti_ref, psem, bsem):
    i = pl.program_id(0)
    q = posq_ref[...]
    qn = jnp.sum(q * q, axis=1, keepdims=True)
    qb = qb_ref[...]
    qidx = i * QB + lax.broadcasted_iota(jnp.int32, (QB, 1), 0)
    bestd_ref[...] = jnp.full((QB, 8), jnp.inf, jnp.float32)
    besti_ref[...] = jnp.zeros((QB, 8), jnp.int32)
    ws = ws_ref[i]
    nt = nt_ref[i]
    lane = lax.broadcasted_iota(jnp.int32, (QB, 8), 1)

    def start_copy(t, slot):
        c0 = pl.multiple_of(ws + t * CT, 128)
        pltpu.make_async_copy(posT_any.at[:, pl.ds(c0, CT)],
                              cpos_ref.at[slot], psem.at[slot]).start()
        pltpu.make_async_copy(brow_any.at[:, pl.ds(c0, CT)],
                              cbat_ref.at[slot], bsem.at[slot]).start()

    start_copy(0, 0)

    def body(t, _):
        slot = lax.rem(t, 2)
        nslot = lax.rem(t + 1, 2)

        @pl.when(t + 1 < nt)
        def _():
            start_copy(t + 1, nslot)

        c0 = pl.multiple_of(ws + t * CT, 128)
        pltpu.make_async_copy(posT_any.at[:, pl.ds(c0, CT)],
                              cpos_ref.at[slot], psem.at[slot]).wait()
        pltpu.make_async_copy(brow_any.at[:, pl.ds(c0, CT)],
                              cbat_ref.at[slot], bsem.at[slot]).wait()
        c = cpos_ref[slot]
        cn = jnp.sum(c * c, axis=0, keepdims=True)
        d = qn + cn - 2.0 * jnp.dot(q, c, preferred_element_type=jnp.float32)
        cbat = cbat_ref[slot]
        lane_ct = lax.broadcasted_iota(jnp.int32, (1, CT), 1)
        cidx = c0 + lane_ct
        key = lax.bitcast_convert_type(jnp.maximum(d, 0.0), jnp.int32)
        key = (key & jnp.int32(-CT)) | lane_ct
        valid = (qb == cbat) & (qidx != cidx)
        key = jnp.where(valid, key, jnp.int32(0x7FC00000))

        bd = bestd_ref[...]
        bi = besti_ref[...]
        for _sel in range(K):
            mk = jnp.min(key, axis=1, keepdims=True)
            key = jnp.where(key == mk, jnp.int32(0x7FC00000), key)
            m = lax.bitcast_convert_type(mk & jnp.int32(-CT), jnp.float32)
            midx = c0 + (mk & jnp.int32(CT - 1))
            bd7 = jnp.where(lane < K, bd, -jnp.inf)
            w = jnp.max(bd7, axis=1, keepdims=True)
            fl = jnp.min(jnp.where(bd7 == w, lane, 8), axis=1, keepdims=True)
            sel = (lane == fl) & (m < w)
            bd = jnp.where(sel, m, bd)
            bi = jnp.where(sel, midx, bi)
        bestd_ref[...] = bd
        besti_ref[...] = bi
        return 0

    lax.fori_loop(0, nt, body, 0)
    nb_ref[...] = jnp.where(lane == K, qidx, besti_ref[...])


def _knn(pos4, batch_col, posTw, broww, ws_al, wcnt):
    nq = NPAD // QB
    return pl.pallas_call(
        _knn_body,
        grid=(nq,),
        in_specs=[
            pl.BlockSpec((QB, 4), lambda i: (i, 0)),
            pl.BlockSpec((QB, 1), lambda i: (i, 0)),
            pl.BlockSpec(memory_space=pltpu.SMEM),
            pl.BlockSpec(memory_space=pltpu.SMEM),
            pl.BlockSpec(memory_space=pl.ANY),
            pl.BlockSpec(memory_space=pl.ANY),
        ],
        out_specs=pl.BlockSpec((QB, 8), lambda i: (i, 0)),
        out_shape=jax.ShapeDtypeStruct((NPAD, 8), jnp.int32),
        scratch_shapes=[
            pltpu.VMEM((2, 4, CT), jnp.float32),
            pltpu.VMEM((2, 1, CT), jnp.int32),
            pltpu.VMEM((QB, 8), jnp.float32),
            pltpu.VMEM((QB, 8), jnp.int32),
            pltpu.SemaphoreType.DMA((2,)),
            pltpu.SemaphoreType.DMA((2,)),
        ],
        compiler_params=pltpu.CompilerParams(
            dimension_semantics=("arbitrary",)),
    )(pos4, batch_col, ws_al, wcnt, posTw, broww)


def _proj_body(x_ref, wl_ref, wr_ref, xl_ref, xr_ref):
    xv = x_ref[...]
    xl_ref[...] = jnp.dot(xv, wl_ref[...], preferred_element_type=jnp.float32)
    xr_ref[...] = jnp.dot(xv, wr_ref[...], preferred_element_type=jnp.float32)


def _proj(xin, Wl, Wr):
    nb2 = NPAD // QB2
    din, cl = Wl.shape
    cr = Wr.shape[1]
    return pl.pallas_call(
        _proj_body,
        grid=(nb2,),
        in_specs=[
            pl.BlockSpec((QB2, din), lambda i: (i, 0)),
            pl.BlockSpec((din, cl), lambda i: (0, 0)),
            pl.BlockSpec((din, cr), lambda i: (0, 0)),
        ],
        out_specs=[
            pl.BlockSpec((QB2, cl), lambda i: (i, 0)),
            pl.BlockSpec((QB2, cr), lambda i: (i, 0)),
        ],
        out_shape=[
            jax.ShapeDtypeStruct((NPAD, cl), jnp.float32),
            jax.ShapeDtypeStruct((NPAD, cr), jnp.float32),
        ],
    )(xin, Wl, Wr)


def _gather_rows(table, idx, etotp, c):
    pw = etotp // SC_NW
    nch = pw // SC_CH
    mesh = plsc.VectorSubcoreMesh(core_axis_name="c", subcore_axis_name="s")

    @functools.partial(
        pl.kernel,
        mesh=mesh,
        out_type=jax.ShapeDtypeStruct((etotp, c), jnp.float32),
        scratch_types=[
            pltpu.VMEM((pw,), jnp.int32),
            pltpu.VMEM((2, SC_CH, c), jnp.float32),
            pltpu.SemaphoreType.DMA((2,)),
        ],
    )
    def gat_gather(table_hbm, idx_hbm, out_hbm, idx_v, rows_v, sem):
        wid = lax.axis_index("s") * SC_NC + lax.axis_index("c")
        base = wid * pw
        pltpu.sync_copy(idx_hbm.at[pl.ds(base, pw)], idx_v)

        def start_g(ci, slot):
            off = ci * SC_CH
            pltpu.make_async_copy(table_hbm.at[idx_v.at[pl.ds(off, SC_CH)]],
                                  rows_v.at[slot], sem.at[slot]).start()

        start_g(0, 0)

        def body(ci, _):
            slot = lax.rem(ci, 2)
            off = ci * SC_CH

            @pl.when(ci + 1 < nch)
            def _():
                start_g(ci + 1, lax.rem(ci + 1, 2))

            pltpu.make_async_copy(table_hbm.at[idx_v.at[pl.ds(off, SC_CH)]],
                                  rows_v.at[slot], sem.at[slot]).wait()
            pltpu.sync_copy(rows_v.at[slot],
                            out_hbm.at[pl.ds(base + off, SC_CH)])
            return 0

        lax.fori_loop(0, nch, body, 0)

    return gat_gather(table, idx)


def _attn_body(g_ref, xr_ref, att_ref, bias_ref, h_ref, *, ch):
    c = NH * ch
    xr = xr_ref[...]
    gs = [g_ref[j][:, :c] for j in range(K)]
    att = att_ref[...]
    S = (lax.broadcasted_iota(jnp.int32, (c, NH), 0) // ch
         == lax.broadcasted_iota(jnp.int32, (c, NH), 1)).astype(jnp.float32)
    ST = (lax.broadcasted_iota(jnp.int32, (NH, c), 0)
          == lax.broadcasted_iota(jnp.int32, (NH, c), 1) // ch
          ).astype(jnp.float32)

    a = []
    for j in range(K):
        v = gs[j] + xr
        e = jnp.where(v >= 0, v, 0.2 * v)
        a.append(jnp.dot(e * att, S, preferred_element_type=jnp.float32))
    amax = a[0]
    for j in range(1, K):
        amax = jnp.maximum(amax, a[j])
    ex = [jnp.exp(a[j] - amax) for j in range(K)]
    denom = ex[0]
    for j in range(1, K):
        denom = denom + ex[j]
    out = jnp.zeros((QB2, c), jnp.float32)
    for j in range(K):
        w = ex[j] / (denom + 1e-16)
        wex = jnp.dot(w, ST, preferred_element_type=jnp.float32)
        out = out + gs[j] * wex
    h_ref[...] = jnp.maximum(out + bias_ref[...], 0.0)


def _attn(g, xr, att, bias, ch):
    nb2 = NPAD // QB2
    c = NH * ch
    gc = g.shape[-1]
    return pl.pallas_call(
        functools.partial(_attn_body, ch=ch),
        grid=(nb2,),
        in_specs=[
            pl.BlockSpec((K, QB2, gc), lambda i: (0, i, 0)),
            pl.BlockSpec((QB2, c), lambda i: (i, 0)),
            pl.BlockSpec((1, c), lambda i: (0, 0)),
            pl.BlockSpec((1, c), lambda i: (0, 0)),
        ],
        out_specs=pl.BlockSpec((QB2, c), lambda i: (i, 0)),
        out_shape=jax.ShapeDtypeStruct((NPAD, c), jnp.float32),
    )(g, xr, att, bias)


BPAD = 56


def _pool_body(h_ref, brow_ref, out_ref, acc_ref, cnt_ref):
    i = pl.program_id(0)
    nb2 = pl.num_programs(0)

    @pl.when(i == 0)
    def _():
        acc_ref[...] = jnp.zeros((BPAD, D), jnp.float32)
        cnt_ref[...] = jnp.zeros((BPAD, 1), jnp.float32)

    br = brow_ref[...]
    oh = (lax.broadcasted_iota(jnp.int32, (BPAD, QB2), 0) == br
          ).astype(jnp.float32)
    acc_ref[...] += jnp.dot(oh, h_ref[...], preferred_element_type=jnp.float32)
    cnt_ref[...] += jnp.sum(oh, axis=1, keepdims=True)

    @pl.when(i == nb2 - 1)
    def _():
        out_ref[...] = acc_ref[...] / jnp.maximum(cnt_ref[...], 1.0)


def _pool(h, batch_row):
    nb2 = NPAD // QB2
    return pl.pallas_call(
        _pool_body,
        grid=(nb2,),
        in_specs=[
            pl.BlockSpec((QB2, D), lambda i: (i, 0)),
            pl.BlockSpec((1, QB2), lambda i: (0, i)),
        ],
        out_specs=pl.BlockSpec((BPAD, D), lambda i: (0, 0)),
        out_shape=jax.ShapeDtypeStruct((BPAD, D), jnp.float32),
        scratch_shapes=[
            pltpu.VMEM((BPAD, D), jnp.float32),
            pltpu.VMEM((BPAD, 1), jnp.float32),
        ],
        compiler_params=pltpu.CompilerParams(
            dimension_semantics=("arbitrary",)),
    )(h, batch_row)


def kernel(x, pos, batch, Wl1, Wr1, att1, bias1, Wl2, Wr2, att2, bias2):
    pad = NPAD - N
    pos4 = jnp.pad(pos, ((0, pad), (0, 1)))
    batch_p = jnp.pad(batch, (0, pad), constant_values=-1)
    batch_col = batch_p.reshape(NPAD, 1)
    batch_row = batch_p.reshape(1, NPAD)
    x_p = jnp.pad(x, ((0, pad), (0, 0)))

    nq = NPAD // QB
    starts = jnp.arange(nq) * QB
    qfirst = batch_p[starts]
    qlast = batch_p[jnp.minimum(starts + QB - 1, N - 1)]
    wstart = jnp.searchsorted(batch, qfirst, side="left").astype(jnp.int32)
    wend = jnp.searchsorted(batch, qlast, side="right").astype(jnp.int32)
    ws_al = (wstart // 128) * 128
    wcnt = (wend - ws_al + CT - 1) // CT

    posTw = jnp.pad(pos4.T, ((0, 0), (0, CT)))
    broww = jnp.pad(batch_row, ((0, 0), (0, CT)), constant_values=-1)
    nb = _knn(pos4, batch_col, posTw, broww, ws_al, wcnt)

    etot = K * NPAD
    idx = nb[:, :K].T.reshape(-1)

    Wl1p = jnp.pad(Wl1, ((0, 0), (0, D - 64)))
    xl1, xr1 = _proj(x_p, Wl1p, Wr1)
    g1 = _gather_rows(xl1, idx, etot, 128).reshape(K, NPAD, 128)
    h1 = _attn(g1, xr1, att1.reshape(1, 64), bias1.reshape(1, 64), 16)

    xl2, xr2 = _proj(h1, Wl2, Wr2)
    g2 = _gather_rows(xl2, idx, etot, 128).reshape(K, NPAD, 128)
    h2 = _attn(g2, xr2, att2.reshape(1, 128), bias2.reshape(1, 128), 32)

    pooled = _pool(h2, batch_row)
    return pooled[:B]
